# baseline (speedup 1.0000x reference)
import functools

import jax
import jax.numpy as jnp
from jax import lax
from jax.experimental import pallas as pl
from jax.experimental.pallas import tpu as pltpu
from jax.experimental.pallas import tpu_sc as plsc

B, N, K = 8, 4096, 20
C_OUT = 64
BN = B * N

NC, NS = 2, 16
NW = NC * NS
NPW = BN // NW
NPS = BN // NS
L = 16

BPT = 2048
NBT = BN // BPT


def _sc_gather_body(i1_hbm, i2_hbm, x_hbm, nr_hbm, o1_hbm, o2_hbm,
                    t1_sh, t2_sh,
                    i1_v, i2_v, x1a_v, x1b_v, x2a_v, x2b_v,
                    r1a_v, r1b_v, r2a_v, r2b_v,
                    rt1a_v, rt1b_v, rt2a_v, rt2b_v,
                    sg1, sg2, sw1, sw2):
    iota = lax.iota(jnp.int32, L)
    cols = [jnp.full((L,), f, jnp.int32) for f in range(8)]

    s = lax.axis_index("s")
    for h in range(2):
        p0 = s * NPS + h * NPW
        b = p0 // N
        nl = p0 % N
        pltpu.sync_copy(x_hbm.at[b, :, pl.ds(nl, NPW)], rt1a_v)
        pltpu.sync_copy(nr_hbm.at[b, :, pl.ds(nl, NPW)], rt2a_v)

        def build_t1(j, _):
            rows = iota + j * L
            sl = pl.ds(j * L, L)
            for f in range(3):
                plsc.store_scatter(r1a_v, [rows, cols[f]], rt1a_v[f, sl])
                plsc.store_scatter(r1a_v, [rows, cols[3 + f]], rt2a_v[f, sl])
                plsc.store_scatter(r2a_v, [rows, cols[f]], rt1a_v[3 + f, sl])
            return 0

        lax.fori_loop(0, NPW // L, build_t1, 0)
        pltpu.sync_copy(r1a_v, t1_sh.at[pl.ds(p0, NPW)])
        pltpu.sync_copy(r2a_v, t2_sh.at[pl.ds(p0, NPW)])
    plsc.subcore_barrier()

    wid = s * NC + lax.axis_index("c")
    n0 = wid * NPW
    pltpu.sync_copy(i1_hbm.at[pl.ds(n0 * K, NPW * K)], i1_v)
    pltpu.sync_copy(i2_hbm.at[pl.ds(n0 * K, NPW * K)], i2_v)
    iota_k = iota * K
    x1 = (x1a_v, x1b_v)
    x2 = (x2a_v, x2b_v)
    r1 = (r1a_v, r1b_v)
    r2 = (r2a_v, r2b_v)
    rt1 = (rt1a_v, rt1b_v)
    rt2 = (rt2a_v, rt2b_v)

    def build_idx(k, xb1, xb2):
        def _(j, _c):
            for u in range(4):
                jj = j * 4 + u
                src = iota_k + (jj * (L * K) + k)
                xb1[pl.ds(jj * L, L)] = plsc.load_gather(i1_v, [src])
                xb2[pl.ds(jj * L, L)] = plsc.load_gather(i2_v, [src])
            return 0

        lax.fori_loop(0, NPW // L // 4, _, 0)

    def fire_gather(xb1, xb2, rb1, rb2):
        return (pltpu.async_copy(t1_sh.at[xb1], rb1, sg1),
                pltpu.async_copy(t2_sh.at[xb2], rb2, sg2))

    def xpose(rb1, rb2, tb1, tb2):
        def _(j, _c):
            for u in range(2):
                jj = j * 2 + u
                rows = iota + jj * L
                sl = pl.ds(jj * L, L)
                for f in range(6):
                    tb1[f, sl] = plsc.load_gather(rb1, [rows, cols[f]])
                for f in range(3):
                    tb2[f, sl] = plsc.load_gather(rb2, [rows, cols[f]])
            return 0

        lax.fori_loop(0, NPW // L // 2, _, 0)

    build_idx(0, x1[0], x2[0])
    g_pend = fire_gather(x1[0], x2[0], r1[0], r2[0])
    w_pend = None
    for k in range(K):
        a = k % 2
        nxt = (k + 1) % 2
        if k < K - 1:
            build_idx(k + 1, x1[nxt], x2[nxt])
            g_next = fire_gather(x1[nxt], x2[nxt], r1[nxt], r2[nxt])
        g_pend[0].wait()
        g_pend[1].wait()
        if k < K - 1:
            g_pend = g_next
        if w_pend is not None:
            w_pend[0].wait()
            w_pend[1].wait()
        xpose(r1[a], r2[a], rt1[a], rt2[a])
        w_pend = (
            pltpu.async_copy(rt1[a], o1_hbm.at[k, :, pl.ds(n0, NPW)], sw1),
            pltpu.async_copy(rt2[a], o2_hbm.at[k, :, pl.ds(n0, NPW)], sw2))
    w_pend[0].wait()
    w_pend[1].wait()


@functools.cache
def _sc_gather():
    f32, i32 = jnp.float32, jnp.int32
    return pl.kernel(
        _sc_gather_body,
        out_type=(jax.ShapeDtypeStruct((K, 6, BN), f32),
                  jax.ShapeDtypeStruct((K, 3, BN), f32)),
        mesh=plsc.VectorSubcoreMesh(
            core_axis_name="c", subcore_axis_name="s", num_cores=NC,
            num_subcores=NS),
        scratch_types=[
            pltpu.MemorySpace.VMEM_SHARED((BN, 8), f32),
            pltpu.MemorySpace.VMEM_SHARED((BN, 8), f32),
            pltpu.VMEM((NPW * K,), i32),
            pltpu.VMEM((NPW * K,), i32),
            pltpu.VMEM((NPW,), i32),
            pltpu.VMEM((NPW,), i32),
            pltpu.VMEM((NPW,), i32),
            pltpu.VMEM((NPW,), i32),
            pltpu.VMEM((NPW, 8), f32),
            pltpu.VMEM((NPW, 8), f32),
            pltpu.VMEM((NPW, 8), f32),
            pltpu.VMEM((NPW, 8), f32),
            pltpu.VMEM((6, NPW), f32),
            pltpu.VMEM((6, NPW), f32),
            pltpu.VMEM((3, NPW), f32),
            pltpu.VMEM((3, NPW), f32),
            pltpu.SemaphoreType.DMA,
            pltpu.SemaphoreType.DMA,
            pltpu.SemaphoreType.DMA,
            pltpu.SemaphoreType.DMA,
        ],
        compiler_params=pltpu.CompilerParams(
            use_tc_tiling_on_sc=False, needs_layout_passes=False),
    )


def _tc_body(g1_ref, g2_ref, x_ref, wa_ref, wb_ref, wd_ref, out_ref):
    cc = jnp.dot(wd_ref[...], x_ref[0], preferred_element_type=jnp.float32)
    acc = None
    for k in range(K):
        hk = (jnp.dot(wa_ref[...], g1_ref[k],
                      preferred_element_type=jnp.float32)
              + jnp.dot(wb_ref[...], g2_ref[k],
                        preferred_element_type=jnp.float32) + cc)
        hk = jnp.where(hk >= 0.0, hk, 0.2 * hk)
        acc = hk if acc is None else jnp.maximum(acc, hk)
    out_ref[0] = acc


def _tc_call(g1, g2, x, wa, wb, wd):
    nb = N // BPT
    return pl.pallas_call(
        _tc_body,
        grid=(NBT,),
        in_specs=[
            pl.BlockSpec((K, 6, BPT), lambda i: (0, 0, i)),
            pl.BlockSpec((K, 3, BPT), lambda i: (0, 0, i)),
            pl.BlockSpec((1, 6, BPT), lambda i: (i // nb, 0, i % nb)),
            pl.BlockSpec((C_OUT, 6), lambda i: (0, 0)),
            pl.BlockSpec((C_OUT, 3), lambda i: (0, 0)),
            pl.BlockSpec((C_OUT, 6), lambda i: (0, 0)),
        ],
        out_specs=pl.BlockSpec(
            (1, C_OUT, BPT), lambda i: (i // nb, 0, i % nb)),
        out_shape=jax.ShapeDtypeStruct((B, C_OUT, N), jnp.float32),
    )(g1, g2, x, wa, wb, wd)


@jax.jit
def _run(x, normalandRGB, idx1, idx2, W):
    g1, g2 = _sc_gather()(idx1, idx2, x, normalandRGB)
    wa = jnp.concatenate([W[:, 0:3], W[:, 12:15]], axis=1)
    wb = W[:, 6:9] + W[:, 9:12]
    wd = jnp.concatenate([W[:, 3:6] - W[:, 0:3], -W[:, 6:9]], axis=1)
    return _tc_call(g1, g2, x, wa, wb, wd)


def kernel(x, normalandRGB, idx1, idx2, W):
    return _run(x, normalandRGB, idx1, idx2, W)


# speedup vs baseline: 1.0172x; 1.0172x over previous
import functools

import jax
import jax.numpy as jnp
from jax import lax
from jax.experimental import pallas as pl
from jax.experimental.pallas import tpu as pltpu
from jax.experimental.pallas import tpu_sc as plsc

B, N, K = 8, 4096, 20
C_OUT = 64
BN = B * N

NC, NS = 2, 16
NW = NC * NS
NPW = BN // NW
NPS = BN // NS
L = 16

BPT = 2048
NBT = BN // BPT


def _sc_gather_body(i1_hbm, i2_hbm, x_hbm, nr_hbm, o1_hbm, o2_hbm,
                    t1_sh, t2_sh,
                    i1_v, i2_v, x1a_v, x1b_v, x2a_v, x2b_v,
                    r1a_v, r1b_v, r2a_v, r2b_v,
                    rt1a_v, rt1b_v, rt2a_v, rt2b_v,
                    sg1, sg2, sw1, sw2):
    iota = lax.iota(jnp.int32, L)
    cols = [jnp.full((L,), f, jnp.int32) for f in range(8)]

    s = lax.axis_index("s")
    for h in range(2):
        p0 = s * NPS + h * NPW
        b = p0 // N
        nl = p0 % N
        pltpu.sync_copy(x_hbm.at[b, :, pl.ds(nl, NPW)], rt1a_v)
        pltpu.sync_copy(nr_hbm.at[b, :, pl.ds(nl, NPW)], rt2a_v)

        def build_t1(j, _):
            rows = iota + j * L
            sl = pl.ds(j * L, L)
            for f in range(3):
                plsc.store_scatter(r1a_v, [rows, cols[f]], rt1a_v[f, sl])
                plsc.store_scatter(r1a_v, [rows, cols[3 + f]], rt2a_v[f, sl])
                plsc.store_scatter(r2a_v, [rows, cols[f]], rt1a_v[3 + f, sl])
            return 0

        lax.fori_loop(0, NPW // L, build_t1, 0)
        pltpu.sync_copy(r1a_v, t1_sh.at[pl.ds(p0, NPW)])
        pltpu.sync_copy(r2a_v, t2_sh.at[pl.ds(p0, NPW)])
    plsc.subcore_barrier()

    wid = s * NC + lax.axis_index("c")
    n0 = wid * NPW
    pltpu.sync_copy(i1_hbm.at[pl.ds(n0 * K, NPW * K)], i1_v)
    pltpu.sync_copy(i2_hbm.at[pl.ds(n0 * K, NPW * K)], i2_v)
    iota_k = iota * K
    x1 = (x1a_v, x1b_v)
    x2 = (x2a_v, x2b_v)
    r1 = (r1a_v, r1b_v)
    r2 = (r2a_v, r2b_v)
    rt1 = (rt1a_v, rt1b_v)
    rt2 = (rt2a_v, rt2b_v)

    def build_idx(k, xb1, xb2):
        def _(j, _c):
            src = iota_k + (j * (L * K) + k)
            xb1[pl.ds(j * L, L)] = plsc.load_gather(i1_v, [src])
            xb2[pl.ds(j * L, L)] = plsc.load_gather(i2_v, [src])
            return 0

        lax.fori_loop(0, NPW // L, _, 0)

    def fire_gather(xb1, xb2, rb1, rb2):
        return (pltpu.async_copy(t1_sh.at[xb1], rb1, sg1),
                pltpu.async_copy(t2_sh.at[xb2], rb2, sg2))

    def xpose(rb1, rb2, tb1, tb2):
        def _(j, _c):
            rows = iota + j * L
            sl = pl.ds(j * L, L)
            for f in range(6):
                tb1[f, sl] = plsc.load_gather(rb1, [rows, cols[f]])
            for f in range(3):
                tb2[f, sl] = plsc.load_gather(rb2, [rows, cols[f]])
            return 0

        lax.fori_loop(0, NPW // L, _, 0)

    build_idx(0, x1[0], x2[0])
    g_pend = fire_gather(x1[0], x2[0], r1[0], r2[0])
    w_pend = None
    for k in range(K):
        a = k % 2
        nxt = (k + 1) % 2
        if k < K - 1:
            build_idx(k + 1, x1[nxt], x2[nxt])
            g_next = fire_gather(x1[nxt], x2[nxt], r1[nxt], r2[nxt])
        g_pend[0].wait()
        g_pend[1].wait()
        if k < K - 1:
            g_pend = g_next
        if w_pend is not None:
            w_pend[0].wait()
            w_pend[1].wait()
        xpose(r1[a], r2[a], rt1[a], rt2[a])
        w_pend = (
            pltpu.async_copy(rt1[a], o1_hbm.at[k, :, pl.ds(n0, NPW)], sw1),
            pltpu.async_copy(rt2[a], o2_hbm.at[k, :, pl.ds(n0, NPW)], sw2))
    w_pend[0].wait()
    w_pend[1].wait()


@functools.cache
def _sc_gather():
    f32, i32 = jnp.float32, jnp.int32
    return pl.kernel(
        _sc_gather_body,
        out_type=(jax.ShapeDtypeStruct((K, 6, BN), f32),
                  jax.ShapeDtypeStruct((K, 3, BN), f32)),
        mesh=plsc.VectorSubcoreMesh(
            core_axis_name="c", subcore_axis_name="s", num_cores=NC,
            num_subcores=NS),
        scratch_types=[
            pltpu.MemorySpace.VMEM_SHARED((BN, 8), f32),
            pltpu.MemorySpace.VMEM_SHARED((BN, 8), f32),
            pltpu.VMEM((NPW * K,), i32),
            pltpu.VMEM((NPW * K,), i32),
            pltpu.VMEM((NPW,), i32),
            pltpu.VMEM((NPW,), i32),
            pltpu.VMEM((NPW,), i32),
            pltpu.VMEM((NPW,), i32),
            pltpu.VMEM((NPW, 8), f32),
            pltpu.VMEM((NPW, 8), f32),
            pltpu.VMEM((NPW, 8), f32),
            pltpu.VMEM((NPW, 8), f32),
            pltpu.VMEM((6, NPW), f32),
            pltpu.VMEM((6, NPW), f32),
            pltpu.VMEM((3, NPW), f32),
            pltpu.VMEM((3, NPW), f32),
            pltpu.SemaphoreType.DMA,
            pltpu.SemaphoreType.DMA,
            pltpu.SemaphoreType.DMA,
            pltpu.SemaphoreType.DMA,
        ],
        compiler_params=pltpu.CompilerParams(
            use_tc_tiling_on_sc=False, needs_layout_passes=False),
    )


def _tc_body(g1_ref, g2_ref, x_ref, wa_ref, wb_ref, wd_ref, out_ref):
    cc = jnp.dot(wd_ref[...], x_ref[0], preferred_element_type=jnp.float32)
    acc = None
    for k in range(K):
        hk = (jnp.dot(wa_ref[...], g1_ref[k],
                      preferred_element_type=jnp.float32)
              + jnp.dot(wb_ref[...], g2_ref[k],
                        preferred_element_type=jnp.float32) + cc)
        hk = jnp.where(hk >= 0.0, hk, 0.2 * hk)
        acc = hk if acc is None else jnp.maximum(acc, hk)
    out_ref[0] = acc


def _tc_call(g1, g2, x, wa, wb, wd):
    nb = N // BPT
    return pl.pallas_call(
        _tc_body,
        grid=(NBT,),
        in_specs=[
            pl.BlockSpec((K, 6, BPT), lambda i: (0, 0, i)),
            pl.BlockSpec((K, 3, BPT), lambda i: (0, 0, i)),
            pl.BlockSpec((1, 6, BPT), lambda i: (i // nb, 0, i % nb)),
            pl.BlockSpec((C_OUT, 6), lambda i: (0, 0)),
            pl.BlockSpec((C_OUT, 3), lambda i: (0, 0)),
            pl.BlockSpec((C_OUT, 6), lambda i: (0, 0)),
        ],
        out_specs=pl.BlockSpec(
            (1, C_OUT, BPT), lambda i: (i // nb, 0, i % nb)),
        out_shape=jax.ShapeDtypeStruct((B, C_OUT, N), jnp.float32),
    )(g1, g2, x, wa, wb, wd)


@jax.jit
def _run(x, normalandRGB, idx1, idx2, W):
    g1, g2 = _sc_gather()(idx1, idx2, x, normalandRGB)
    wa = jnp.concatenate([W[:, 0:3], W[:, 12:15]], axis=1)
    wb = W[:, 6:9] + W[:, 9:12]
    wd = jnp.concatenate([W[:, 3:6] - W[:, 0:3], -W[:, 6:9]], axis=1)
    return _tc_call(g1, g2, x, wa, wb, wd)


def kernel(x, normalandRGB, idx1, idx2, W):
    return _run(x, normalandRGB, idx1, idx2, W)


# speedup vs baseline: 1.1579x; 1.1384x over previous
import functools

import jax
import jax.numpy as jnp
from jax import lax
from jax.experimental import pallas as pl
from jax.experimental.pallas import tpu as pltpu
from jax.experimental.pallas import tpu_sc as plsc

B, N, K = 8, 4096, 20
C_OUT = 64
BN = B * N

NC, NS = 2, 16
NW = NC * NS
NPW = BN // NW
NPS = BN // NS
L = 16

BPT = 2048
NBT = BN // BPT


def _sc_gather_body(i1_hbm, i2_hbm, x_hbm, nr_hbm, o_hbm,
                    t1_sh, t2_sh,
                    i1_v, i2_v, x1a_v, x1b_v, x2a_v, x2b_v,
                    r1a_v, r1b_v, r2a_v, r2b_v,
                    rt1a_v, rt1b_v,
                    sg1, sg2, sw1):
    iota = lax.iota(jnp.int32, L)
    cols = [jnp.full((L,), f, jnp.int32) for f in range(8)]

    s = lax.axis_index("s")
    for h in range(2):
        p0 = s * NPS + h * NPW
        b = p0 // N
        nl = p0 % N
        pltpu.sync_copy(x_hbm.at[b, :, pl.ds(nl, NPW)], rt1b_v.at[0:6])
        pltpu.sync_copy(nr_hbm.at[b, :, pl.ds(nl, NPW)], rt1a_v.at[0:3])

        def build_t1(j, _):
            rows = iota + j * L
            sl = pl.ds(j * L, L)
            for f in range(3):
                plsc.store_scatter(r1a_v, [rows, cols[f]], rt1b_v[f, sl])
                plsc.store_scatter(r1a_v, [rows, cols[3 + f]], rt1a_v[f, sl])
                plsc.store_scatter(r2a_v, [rows, cols[f]], rt1b_v[3 + f, sl])
            return 0

        lax.fori_loop(0, NPW // L, build_t1, 0)
        pltpu.sync_copy(r1a_v, t1_sh.at[pl.ds(p0, NPW)])
        pltpu.sync_copy(r2a_v, t2_sh.at[pl.ds(p0, NPW)])
    plsc.subcore_barrier()

    wid = s * NC + lax.axis_index("c")
    n0 = wid * NPW
    pltpu.sync_copy(i1_hbm.at[pl.ds(n0 * K, NPW * K)], i1_v)
    pltpu.sync_copy(i2_hbm.at[pl.ds(n0 * K, NPW * K)], i2_v)
    iota_k = iota * K
    x1 = (x1a_v, x1b_v)
    x2 = (x2a_v, x2b_v)
    r1 = (r1a_v, r1b_v)
    r2 = (r2a_v, r2b_v)
    rt = (rt1a_v, rt1b_v)

    def build_idx(k, xb1, xb2):
        def _(j, _c):
            src = iota_k + (j * (L * K) + k)
            xb1[pl.ds(j * L, L)] = plsc.load_gather(i1_v, [src])
            xb2[pl.ds(j * L, L)] = plsc.load_gather(i2_v, [src])
            return 0

        lax.fori_loop(0, NPW // L, _, 0)

    def fire_gather(xb1, xb2, rb1, rb2):
        return (pltpu.async_copy(t1_sh.at[xb1], rb1, sg1),
                pltpu.async_copy(t2_sh.at[xb2], rb2, sg2))

    def xpose(rb1, rb2, tb):
        def _(j, _c):
            rows = iota + j * L
            sl = pl.ds(j * L, L)
            for f in range(6):
                tb[f, sl] = plsc.load_gather(rb1, [rows, cols[f]])
            for f in range(3):
                tb[6 + f, sl] = plsc.load_gather(rb2, [rows, cols[f]])
            return 0

        lax.fori_loop(0, NPW // L, _, 0)

    build_idx(0, x1[0], x2[0])
    g_pend = fire_gather(x1[0], x2[0], r1[0], r2[0])
    w_pend = None
    for k in range(K):
        a = k % 2
        nxt = (k + 1) % 2
        if k < K - 1:
            build_idx(k + 1, x1[nxt], x2[nxt])
            g_next = fire_gather(x1[nxt], x2[nxt], r1[nxt], r2[nxt])
        g_pend[0].wait()
        g_pend[1].wait()
        if k < K - 1:
            g_pend = g_next
        if w_pend is not None:
            w_pend.wait()
        xpose(r1[a], r2[a], rt[a])
        w_pend = pltpu.async_copy(
            rt[a], o_hbm.at[k, :, pl.ds(n0, NPW)], sw1)
    w_pend.wait()


@functools.cache
def _sc_gather():
    f32, i32 = jnp.float32, jnp.int32
    return pl.kernel(
        _sc_gather_body,
        out_type=jax.ShapeDtypeStruct((K, 9, BN), f32),
        mesh=plsc.VectorSubcoreMesh(
            core_axis_name="c", subcore_axis_name="s", num_cores=NC,
            num_subcores=NS),
        scratch_types=[
            pltpu.MemorySpace.VMEM_SHARED((BN, 8), f32),
            pltpu.MemorySpace.VMEM_SHARED((BN, 8), f32),
            pltpu.VMEM((NPW * K,), i32),
            pltpu.VMEM((NPW * K,), i32),
            pltpu.VMEM((NPW,), i32),
            pltpu.VMEM((NPW,), i32),
            pltpu.VMEM((NPW,), i32),
            pltpu.VMEM((NPW,), i32),
            pltpu.VMEM((NPW, 8), f32),
            pltpu.VMEM((NPW, 8), f32),
            pltpu.VMEM((NPW, 8), f32),
            pltpu.VMEM((NPW, 8), f32),
            pltpu.VMEM((9, NPW), f32),
            pltpu.VMEM((9, NPW), f32),
            pltpu.SemaphoreType.DMA,
            pltpu.SemaphoreType.DMA,
            pltpu.SemaphoreType.DMA,
        ],
        compiler_params=pltpu.CompilerParams(
            use_tc_tiling_on_sc=False, needs_layout_passes=False),
    )


def _tc_body(g_ref, x_ref, wab_ref, wd_ref, out_ref):
    cc = jnp.dot(wd_ref[...], x_ref[0], preferred_element_type=jnp.float32)
    acc = None
    for k in range(K):
        hk = jnp.dot(wab_ref[...], g_ref[k],
                     preferred_element_type=jnp.float32) + cc
        hk = jnp.maximum(hk, 0.2 * hk)
        acc = hk if acc is None else jnp.maximum(acc, hk)
    out_ref[0] = acc


def _tc_call(g, x, wab, wd):
    nb = N // BPT
    return pl.pallas_call(
        _tc_body,
        grid=(NBT,),
        in_specs=[
            pl.BlockSpec((K, 9, BPT), lambda i: (0, 0, i)),
            pl.BlockSpec((1, 6, BPT), lambda i: (i // nb, 0, i % nb)),
            pl.BlockSpec((C_OUT, 9), lambda i: (0, 0)),
            pl.BlockSpec((C_OUT, 6), lambda i: (0, 0)),
        ],
        out_specs=pl.BlockSpec(
            (1, C_OUT, BPT), lambda i: (i // nb, 0, i % nb)),
        out_shape=jax.ShapeDtypeStruct((B, C_OUT, N), jnp.float32),
    )(g, x, wab, wd)


@jax.jit
def _run(x, normalandRGB, idx1, idx2, W):
    g = _sc_gather()(idx1, idx2, x, normalandRGB)
    wab = jnp.concatenate(
        [W[:, 0:3], W[:, 12:15], W[:, 6:9] + W[:, 9:12]], axis=1)
    wd = jnp.concatenate([W[:, 3:6] - W[:, 0:3], -W[:, 6:9]], axis=1)
    return _tc_call(g, x, wab, wd)


def kernel(x, normalandRGB, idx1, idx2, W):
    return _run(x, normalandRGB, idx1, idx2, W)


# speedup vs baseline: 1.1655x; 1.0065x over previous
import functools

import jax
import jax.numpy as jnp
from jax import lax
from jax.experimental import pallas as pl
from jax.experimental.pallas import tpu as pltpu
from jax.experimental.pallas import tpu_sc as plsc

B, N, K = 8, 4096, 20
C_OUT = 64
BN = B * N

NC, NS = 2, 16
NW = NC * NS
NPW = BN // NW
NPS = BN // NS
L = 16

BPT = 4096
NBT = BN // BPT


def _sc_gather_body(i1_hbm, i2_hbm, x_hbm, nr_hbm, o_hbm,
                    t1_sh, t2_sh,
                    i1_v, i2_v, x1a_v, x1b_v, x2a_v, x2b_v,
                    r1a_v, r1b_v, r2a_v, r2b_v,
                    rt1a_v, rt1b_v,
                    sg1, sg2, sw1):
    iota = lax.iota(jnp.int32, L)
    cols = [jnp.full((L,), f, jnp.int32) for f in range(8)]

    s = lax.axis_index("s")
    for h in range(2):
        p0 = s * NPS + h * NPW
        b = p0 // N
        nl = p0 % N
        pltpu.sync_copy(x_hbm.at[b, :, pl.ds(nl, NPW)], rt1b_v.at[0:6])
        pltpu.sync_copy(nr_hbm.at[b, :, pl.ds(nl, NPW)], rt1a_v.at[0:3])

        def build_t1(j, _):
            rows = iota + j * L
            sl = pl.ds(j * L, L)
            for f in range(3):
                plsc.store_scatter(r1a_v, [rows, cols[f]], rt1b_v[f, sl])
                plsc.store_scatter(r1a_v, [rows, cols[3 + f]], rt1a_v[f, sl])
                plsc.store_scatter(r2a_v, [rows, cols[f]], rt1b_v[3 + f, sl])
            return 0

        lax.fori_loop(0, NPW // L, build_t1, 0)
        pltpu.sync_copy(r1a_v, t1_sh.at[pl.ds(p0, NPW)])
        pltpu.sync_copy(r2a_v, t2_sh.at[pl.ds(p0, NPW)])
    plsc.subcore_barrier()

    wid = s * NC + lax.axis_index("c")
    n0 = wid * NPW
    pltpu.sync_copy(i1_hbm.at[pl.ds(n0 * K, NPW * K)], i1_v)
    pltpu.sync_copy(i2_hbm.at[pl.ds(n0 * K, NPW * K)], i2_v)
    iota_k = iota * K
    x1 = (x1a_v, x1b_v)
    x2 = (x2a_v, x2b_v)
    r1 = (r1a_v, r1b_v)
    r2 = (r2a_v, r2b_v)
    rt = (rt1a_v, rt1b_v)

    def build_idx(k, xb1, xb2):
        def _(j, _c):
            src = iota_k + (j * (L * K) + k)
            xb1[pl.ds(j * L, L)] = plsc.load_gather(i1_v, [src])
            xb2[pl.ds(j * L, L)] = plsc.load_gather(i2_v, [src])
            return 0

        lax.fori_loop(0, NPW // L, _, 0)

    def fire_gather(xb1, xb2, rb1, rb2):
        return (pltpu.async_copy(t1_sh.at[xb1], rb1, sg1),
                pltpu.async_copy(t2_sh.at[xb2], rb2, sg2))

    def xpose(rb1, rb2, tb):
        def _(j, _c):
            rows = iota + j * L
            sl = pl.ds(j * L, L)
            for f in range(6):
                tb[f, sl] = plsc.load_gather(rb1, [rows, cols[f]])
            for f in range(3):
                tb[6 + f, sl] = plsc.load_gather(rb2, [rows, cols[f]])
            return 0

        lax.fori_loop(0, NPW // L, _, 0)

    build_idx(0, x1[0], x2[0])
    g_pend = fire_gather(x1[0], x2[0], r1[0], r2[0])
    w_pend = None
    for k in range(K):
        a = k % 2
        nxt = (k + 1) % 2
        if k < K - 1:
            build_idx(k + 1, x1[nxt], x2[nxt])
            g_next = fire_gather(x1[nxt], x2[nxt], r1[nxt], r2[nxt])
        g_pend[0].wait()
        g_pend[1].wait()
        if k < K - 1:
            g_pend = g_next
        if w_pend is not None:
            w_pend.wait()
        xpose(r1[a], r2[a], rt[a])
        w_pend = pltpu.async_copy(
            rt[a], o_hbm.at[k, :, pl.ds(n0, NPW)], sw1)
    w_pend.wait()


@functools.cache
def _sc_gather():
    f32, i32 = jnp.float32, jnp.int32
    return pl.kernel(
        _sc_gather_body,
        out_type=jax.ShapeDtypeStruct((K, 9, BN), f32),
        mesh=plsc.VectorSubcoreMesh(
            core_axis_name="c", subcore_axis_name="s", num_cores=NC,
            num_subcores=NS),
        scratch_types=[
            pltpu.MemorySpace.VMEM_SHARED((BN, 8), f32),
            pltpu.MemorySpace.VMEM_SHARED((BN, 8), f32),
            pltpu.VMEM((NPW * K,), i32),
            pltpu.VMEM((NPW * K,), i32),
            pltpu.VMEM((NPW,), i32),
            pltpu.VMEM((NPW,), i32),
            pltpu.VMEM((NPW,), i32),
            pltpu.VMEM((NPW,), i32),
            pltpu.VMEM((NPW, 8), f32),
            pltpu.VMEM((NPW, 8), f32),
            pltpu.VMEM((NPW, 8), f32),
            pltpu.VMEM((NPW, 8), f32),
            pltpu.VMEM((9, NPW), f32),
            pltpu.VMEM((9, NPW), f32),
            pltpu.SemaphoreType.DMA,
            pltpu.SemaphoreType.DMA,
            pltpu.SemaphoreType.DMA,
        ],
        compiler_params=pltpu.CompilerParams(
            use_tc_tiling_on_sc=False, needs_layout_passes=False),
    )


def _tc_body(g_ref, x_ref, wab_ref, wd_ref, out_ref):
    cc = jnp.dot(wd_ref[...], x_ref[0], preferred_element_type=jnp.float32)
    acc = None
    for k in range(K):
        hk = jnp.dot(wab_ref[...], g_ref[k],
                     preferred_element_type=jnp.float32) + cc
        hk = jnp.maximum(hk, 0.2 * hk)
        acc = hk if acc is None else jnp.maximum(acc, hk)
    out_ref[0] = acc


def _tc_call(g, x, wab, wd):
    nb = N // BPT
    return pl.pallas_call(
        _tc_body,
        grid=(NBT,),
        in_specs=[
            pl.BlockSpec((K, 9, BPT), lambda i: (0, 0, i)),
            pl.BlockSpec((1, 6, BPT), lambda i: (i // nb, 0, i % nb)),
            pl.BlockSpec((C_OUT, 9), lambda i: (0, 0)),
            pl.BlockSpec((C_OUT, 6), lambda i: (0, 0)),
        ],
        out_specs=pl.BlockSpec(
            (1, C_OUT, BPT), lambda i: (i // nb, 0, i % nb)),
        out_shape=jax.ShapeDtypeStruct((B, C_OUT, N), jnp.float32),
    )(g, x, wab, wd)


@jax.jit
def _run(x, normalandRGB, idx1, idx2, W):
    g = _sc_gather()(idx1, idx2, x, normalandRGB)
    wab = jnp.concatenate(
        [W[:, 0:3], W[:, 12:15], W[:, 6:9] + W[:, 9:12]], axis=1)
    wd = jnp.concatenate([W[:, 3:6] - W[:, 0:3], -W[:, 6:9]], axis=1)
    return _tc_call(g, x, wab, wd)


def kernel(x, normalandRGB, idx1, idx2, W):
    return _run(x, normalandRGB, idx1, idx2, W)
